# baseline (speedup 1.0000x reference)
import jax
import jax.numpy as jnp
from jax.experimental import pallas as pl

_BN = 1024
_BK = 256


def _fused_kernel(x_ref, g_ref, c_ref, o_ref):
    k = pl.program_id(1)
    x = x_ref[...]
    g = g_ref[...]
    xx = jnp.sum(x * x, axis=1, keepdims=True)
    gg = jnp.sum(g * g, axis=1)
    xg = jax.lax.dot_general(x, g, (((1,), (1,)), ((), ())),
                             preferred_element_type=jnp.float32)
    sq = jnp.maximum(xx - 2.0 * xg + gg[None, :], 0.0)
    kt = jnp.exp(-0.5 * sq)
    contrib = jnp.dot(kt, c_ref[...], preferred_element_type=jnp.float32)

    @pl.when(k == 0)
    def _init():
        o_ref[...] = contrib

    @pl.when(k != 0)
    def _acc():
        o_ref[...] += contrib


def kernel(input, sparse_grid, chol_inv):
    n, d = input.shape
    m = sparse_grid.shape[0]
    grid = (n // _BN, m // _BK)
    return pl.pallas_call(
        _fused_kernel,
        grid=grid,
        in_specs=[
            pl.BlockSpec((_BN, d), lambda i, k: (i, 0)),
            pl.BlockSpec((_BK, d), lambda i, k: (k, 0)),
            pl.BlockSpec((_BK, m), lambda i, k: (k, 0)),
        ],
        out_specs=pl.BlockSpec((_BN, m), lambda i, k: (i, 0)),
        out_shape=jax.ShapeDtypeStruct((n, m), jnp.float32),
    )(input, sparse_grid, chol_inv)


# speedup vs baseline: 1.1167x; 1.1167x over previous
import jax
import jax.numpy as jnp
from jax.experimental import pallas as pl

_BN = 1024
_BK = 512


def _fused_kernel(x_ref, g_ref, c_ref, o_ref):
    k = pl.program_id(1)
    x = x_ref[...]
    g = g_ref[...]
    xx = jnp.sum(x * x, axis=1, keepdims=True)
    gg = jnp.sum(g * g, axis=1)
    xg = jax.lax.dot_general(x, g, (((1,), (1,)), ((), ())),
                             preferred_element_type=jnp.float32)
    sq = jnp.maximum(xx - 2.0 * xg + gg[None, :], 0.0)
    kt = jnp.exp(-0.5 * sq).astype(jnp.bfloat16)
    contrib = jnp.dot(kt, c_ref[...], preferred_element_type=jnp.float32)

    @pl.when(k == 0)
    def _init():
        o_ref[...] = contrib

    @pl.when(k != 0)
    def _acc():
        o_ref[...] += contrib


def kernel(input, sparse_grid, chol_inv):
    n, d = input.shape
    m = sparse_grid.shape[0]
    grid = (n // _BN, m // _BK)
    c_bf = chol_inv.astype(jnp.bfloat16)
    return pl.pallas_call(
        _fused_kernel,
        grid=grid,
        in_specs=[
            pl.BlockSpec((_BN, d), lambda i, k: (i, 0)),
            pl.BlockSpec((_BK, d), lambda i, k: (k, 0)),
            pl.BlockSpec((_BK, m), lambda i, k: (k, 0)),
        ],
        out_specs=pl.BlockSpec((_BN, m), lambda i, k: (i, 0)),
        out_shape=jax.ShapeDtypeStruct((n, m), jnp.float32),
    )(input, sparse_grid, c_bf)


# speedup vs baseline: 1.2066x; 1.0804x over previous
import jax
import jax.numpy as jnp
import numpy as np
from jax.experimental import pallas as pl
from jax.experimental.pallas import tpu as pltpu

_BN = 1024
_BB = 512


def _kern(jm_ref, km_ref, x_ref, g_ref, c_ref, o_ref, kt_ref):
    t = pl.program_id(1)
    j = jm_ref[t]
    k = km_ref[t]

    @pl.when(j == 0)
    def _compute_kt():
        x = x_ref[...]
        g = g_ref[...]
        xx = jnp.sum(x * x, axis=1, keepdims=True)
        gg = jnp.sum(g * g, axis=1)
        xg = jax.lax.dot_general(x, g, (((1,), (1,)), ((), ())),
                                 preferred_element_type=jnp.float32)
        sq = jnp.maximum(xx - 2.0 * xg + gg[None, :], 0.0)
        kt_ref[k] = jnp.exp(-0.5 * sq).astype(jnp.bfloat16)

    contrib = jnp.dot(kt_ref[k], c_ref[...],
                      preferred_element_type=jnp.float32)

    @pl.when(k == j)
    def _init():
        o_ref[...] = contrib

    @pl.when(k != j)
    def _acc():
        o_ref[...] += contrib


def kernel(input, sparse_grid, chol_inv):
    n, d = input.shape
    m = sparse_grid.shape[0]
    nb = m // _BB
    js, ks = [], []
    for j in range(nb):
        for k in range(j, nb):
            js.append(j)
            ks.append(k)
    jm = jnp.asarray(np.array(js, dtype=np.int32))
    km = jnp.asarray(np.array(ks, dtype=np.int32))
    c_bf = chol_inv.astype(jnp.bfloat16)

    grid_spec = pltpu.PrefetchScalarGridSpec(
        num_scalar_prefetch=2,
        grid=(n // _BN, len(js)),
        in_specs=[
            pl.BlockSpec((_BN, d), lambda i, t, jm, km: (i, 0)),
            pl.BlockSpec((_BB, d), lambda i, t, jm, km: (km[t], 0)),
            pl.BlockSpec((_BB, _BB), lambda i, t, jm, km: (km[t], jm[t])),
        ],
        out_specs=pl.BlockSpec((_BN, _BB), lambda i, t, jm, km: (i, jm[t])),
        scratch_shapes=[pltpu.VMEM((nb, _BN, _BB), jnp.bfloat16)],
    )
    return pl.pallas_call(
        _kern,
        grid_spec=grid_spec,
        out_shape=jax.ShapeDtypeStruct((n, m), jnp.float32),
    )(jm, km, input, sparse_grid, c_bf)


# speedup vs baseline: 1.8438x; 1.5281x over previous
import jax
import jax.numpy as jnp
from jax.experimental import pallas as pl
from jax.experimental.pallas import tpu as pltpu

_BN = 1024
_BB = 512


def _kern(x_ref, g_ref, c_ref, o_ref, kt_ref):
    j = pl.program_id(1)
    nb = c_ref.shape[1] // _BB

    @pl.when(j == 0)
    def _compute_kt():
        x = x_ref[...]
        xx = jnp.sum(x * x, axis=1, keepdims=True)
        for k in range(nb):
            g = g_ref[k * _BB:(k + 1) * _BB, :]
            gg = jnp.sum(g * g, axis=1)
            xg = jax.lax.dot_general(x, g, (((1,), (1,)), ((), ())),
                                     preferred_element_type=jnp.float32)
            sq = jnp.maximum(xx - 2.0 * xg + gg[None, :], 0.0)
            kt_ref[:, k * _BB:(k + 1) * _BB] = (
                jnp.exp(-0.5 * sq).astype(jnp.bfloat16))

    for jj in range(nb):
        @pl.when(j == jj)
        def _panel(jj=jj):
            lo = jj * _BB
            o_ref[...] = jnp.dot(
                kt_ref[:, lo:],
                c_ref[lo:, lo:lo + _BB],
                preferred_element_type=jnp.float32,
            )


def kernel(input, sparse_grid, chol_inv):
    n, d = input.shape
    m = sparse_grid.shape[0]
    c_bf = chol_inv.astype(jnp.bfloat16)
    return pl.pallas_call(
        _kern,
        grid=(n // _BN, m // _BB),
        in_specs=[
            pl.BlockSpec((_BN, d), lambda i, j: (i, 0)),
            pl.BlockSpec((m, d), lambda i, j: (0, 0)),
            pl.BlockSpec((m, m), lambda i, j: (0, 0)),
        ],
        out_specs=pl.BlockSpec((_BN, _BB), lambda i, j: (i, j)),
        out_shape=jax.ShapeDtypeStruct((n, m), jnp.float32),
        scratch_shapes=[pltpu.VMEM((_BN, m), jnp.bfloat16)],
    )(input, sparse_grid, c_bf)


# speedup vs baseline: 1.9839x; 1.0760x over previous
import jax
import jax.numpy as jnp
from jax.experimental import pallas as pl
from jax.experimental.pallas import tpu as pltpu

_BN = 512
_BB = 512


def _kern(x_ref, g_ref, c_ref, o_ref, kt_ref, cb_ref):
    i = pl.program_id(0)
    j = pl.program_id(1)
    nb = cb_ref.shape[1] // _BB

    @pl.when(j == 0)
    def _compute_kt():
        x = x_ref[...]
        xx = jnp.sum(x * x, axis=1, keepdims=True)
        for k in range(nb):
            g = g_ref[k * _BB:(k + 1) * _BB, :]
            gg = jnp.sum(g * g, axis=1)
            xg = jax.lax.dot_general(x, g, (((1,), (1,)), ((), ())),
                                     preferred_element_type=jnp.float32)
            sq = jnp.maximum(xx - 2.0 * xg + gg[None, :], 0.0)
            kt_ref[:, k * _BB:(k + 1) * _BB] = (
                jnp.exp(-0.5 * sq).astype(jnp.bfloat16))

    for jj in range(nb):
        @pl.when(j == jj)
        def _panel(jj=jj):
            lo = jj * _BB

            @pl.when(i == 0)
            def _cast_panel():
                cb_ref[lo:, lo:lo + _BB] = (
                    c_ref[lo:, :].astype(jnp.bfloat16))

            o_ref[...] = jnp.dot(
                kt_ref[:, lo:],
                cb_ref[lo:, lo:lo + _BB],
                preferred_element_type=jnp.float32,
            )


def kernel(input, sparse_grid, chol_inv):
    n, d = input.shape
    m = sparse_grid.shape[0]
    nb = m // _BB

    def c_map(i, j):
        return (0, jnp.where(i == 0, j, nb - 1))

    return pl.pallas_call(
        _kern,
        grid=(n // _BN, nb),
        in_specs=[
            pl.BlockSpec((_BN, d), lambda i, j: (i, 0)),
            pl.BlockSpec((m, d), lambda i, j: (0, 0)),
            pl.BlockSpec((m, _BB), c_map),
        ],
        out_specs=pl.BlockSpec((_BN, _BB), lambda i, j: (i, j)),
        out_shape=jax.ShapeDtypeStruct((n, m), jnp.float32),
        scratch_shapes=[
            pltpu.VMEM((_BN, m), jnp.bfloat16),
            pltpu.VMEM((m, m), jnp.bfloat16),
        ],
    )(input, sparse_grid, chol_inv)
